# baseline (speedup 1.0000x reference)
import functools

import jax
import jax.numpy as jnp
from jax import lax
from jax.experimental import pallas as pl
from jax.experimental.pallas import tpu as pltpu
from jax.experimental.pallas import tpu_sc as plsc

N = 10000
E = 320000
D = 128
H = 128
C = 64

NC = 2
NS = 16
NW = NC * NS
LANES = 16

NPAD = 10240
K = 128
NCHUNKS = E // K
CHUNKS_PER_W = -(-NCHUNKS // NW)
CMAX = 80
NCHUNKS_PAD = NW * CMAX
SUP = 40
KP = 400
EPW = E // NW
PBLOCKS = EPW // KP
ROWS_PER_TILE = NPAD // NS
BR = 1024


def _wid():
    c = lax.axis_index("c")
    s = lax.axis_index("s")
    return s * NC + c, c, s


def _prep_body(row_hbm, col_hbm, degp_hbm,
               degall_sp, row_a, col_a, row_b, col_b, sema, semb,
               deg_v, tmp_v, acc_v):
    wid, c, s = _wid()
    zeros16 = jnp.zeros((LANES,), jnp.float32)
    ones16 = jnp.ones((LANES,), jnp.float32)
    e0 = wid * EPW
    bufs = [(row_a, col_a, sema), (row_b, col_b, semb)]

    def fire(b, rv, cv, sem):
        @pl.when(b < PBLOCKS)
        def _():
            base = e0 + b * KP
            pltpu.async_copy(row_hbm.at[pl.ds(base, KP)], rv, sem)
            pltpu.async_copy(col_hbm.at[pl.ds(base, KP)], cv, sem)

    def consume(b, rv, cv, sem):
        @pl.when(b < PBLOCKS)
        def _():
            base = e0 + b * KP
            pltpu.make_async_copy(row_hbm.at[pl.ds(base, KP)], rv, sem).wait()
            pltpu.make_async_copy(col_hbm.at[pl.ds(base, KP)], cv, sem).wait()
            for j in range(KP // LANES):
                sl = pl.ds(j * LANES, LANES)
                r = rv[sl]
                co = cv[sl]
                cp = jnp.where(r == co, N, co)
                plsc.addupdate_scatter(deg_v, [cp], ones16)

    @pl.loop(0, NPAD // LANES)
    def _(j):
        deg_v[pl.ds(j * LANES, LANES)] = zeros16

    fire(0, *bufs[0])

    @pl.loop(0, PBLOCKS + 1, step=2)
    def _(t):
        fire(t + 1, *bufs[1])
        consume(t, *bufs[0])
        fire(t + 2, *bufs[0])
        consume(t + 1, *bufs[1])

    pltpu.sync_copy(deg_v, degall_sp.at[s])
    plsc.subcore_barrier()

    @pl.loop(0, ROWS_PER_TILE // LANES)
    def _(j):
        acc_v[pl.ds(j * LANES, LANES)] = zeros16

    @pl.loop(0, NS)
    def _(t):
        pltpu.sync_copy(degall_sp.at[t, pl.ds(s * ROWS_PER_TILE, ROWS_PER_TILE)],
                        tmp_v)

        @pl.loop(0, ROWS_PER_TILE // LANES)
        def _(j):
            sl = pl.ds(j * LANES, LANES)
            acc_v[sl] = acc_v[sl] + tmp_v[sl]

    pltpu.sync_copy(acc_v, degp_hbm.at[c, pl.ds(s * ROWS_PER_TILE, ROWS_PER_TILE)])


def _sc_mesh():
    return plsc.VectorSubcoreMesh(core_axis_name="c", subcore_axis_name="s",
                                  num_cores=NC, num_subcores=NS)


@functools.cache
def _build_prep():
    return functools.partial(
        pl.kernel,
        out_type=jax.ShapeDtypeStruct((NC, NPAD), jnp.float32),
        mesh=_sc_mesh(),
        compiler_params=pltpu.CompilerParams(needs_layout_passes=False),
        scratch_types=[
            pltpu.VMEM_SHARED((NS, NPAD), jnp.float32),
            pltpu.VMEM((KP,), jnp.int32),
            pltpu.VMEM((KP,), jnp.int32),
            pltpu.VMEM((KP,), jnp.int32),
            pltpu.VMEM((KP,), jnp.int32),
            pltpu.SemaphoreType.DMA,
            pltpu.SemaphoreType.DMA,
            pltpu.VMEM((NPAD,), jnp.float32),
            pltpu.VMEM((ROWS_PER_TILE,), jnp.float32),
            pltpu.VMEM((ROWS_PER_TILE,), jnp.float32),
        ],
    )(_prep_body)


def _scatter_body(y_hbm, row2_hbm, col2_hbm, zero_hbm, out_hbm,
                  acc_sp, ridx2, cidx2, rows0, rows1, sem0, sem1, ssem0, ssem1):
    wid, c, s = _wid()
    r0 = s * ROWS_PER_TILE
    rows = [rows0, rows1]
    sems = [sem0, sem1]
    ssems = [ssem0, ssem1]
    start = CMAX * wid
    nch = jnp.minimum(CMAX, NCHUNKS - start)

    pltpu.sync_copy(zero_hbm, rows0)
    for i in range(ROWS_PER_TILE // K):
        pltpu.async_copy(rows0, acc_sp.at[pl.ds(r0 + i * K, K)], sem0)
    for i in range(ROWS_PER_TILE // K):
        pltpu.make_async_copy(rows0, acc_sp.at[pl.ds(r0 + i * K, K)],
                              sem0).wait()

    plsc.subcore_barrier()

    KH = K // 2

    def fire(u, j, b, drain=True):
        q = u * SUP + j

        @pl.when(q < nch)
        def _():
            if drain:
                pltpu.make_async_copy(rows[b], acc_sp.at[cidx2.at[j]],
                                      ssems[b]).wait()
            pltpu.async_copy(y_hbm.at[ridx2.at[j, pl.ds(0, KH)]],
                             rows[b].at[pl.ds(0, KH)], sems[b])
            pltpu.async_copy(y_hbm.at[ridx2.at[j, pl.ds(KH, KH)]],
                             rows[b].at[pl.ds(KH, KH)], sems[b])

    def consume(u, j, b):
        q = u * SUP + j

        @pl.when(q < nch)
        def _():
            pltpu.make_async_copy(y_hbm.at[ridx2.at[j, pl.ds(0, KH)]],
                                  rows[b].at[pl.ds(0, KH)], sems[b]).wait()
            pltpu.make_async_copy(y_hbm.at[ridx2.at[j, pl.ds(KH, KH)]],
                                  rows[b].at[pl.ds(KH, KH)], sems[b]).wait()
            pltpu.async_copy(rows[b], acc_sp.at[cidx2.at[j]], ssems[b],
                             add=True)

    @pl.loop(0, CMAX // SUP)
    def _(u):
        @pl.when(u * SUP < nch)
        def _():
            sl_u = pl.ds(start + u * SUP, SUP)
            pltpu.sync_copy(row2_hbm.at[sl_u], ridx2)
            pltpu.sync_copy(col2_hbm.at[sl_u], cidx2)

            @pl.loop(0, SUP)
            def _(q):
                for j in range(K // LANES):
                    sl = pl.ds(j * LANES, LANES)
                    r = ridx2[q, sl]
                    cv = cidx2[q, sl]
                    cidx2[q, sl] = jnp.where(r == cv, N, cv)

        fire(u, 0, 0, drain=False)
        fire(u, 1, 1, drain=False)
        for j in range(0, SUP - 2, 2):
            consume(u, j, 0)
            consume(u, j + 1, 1)
            fire(u, j + 2, 0)
            fire(u, j + 3, 1)
        consume(u, SUP - 2, 0)
        consume(u, SUP - 1, 1)
        for j in range(SUP):
            q = u * SUP + j
            pend = (q < nch) if j >= SUP - 2 else ((q < nch) & (q + 2 >= nch))

            @pl.when(pend)
            def _(j=j):
                pltpu.make_async_copy(rows[j % 2], acc_sp.at[cidx2.at[j]],
                                      ssems[j % 2]).wait()

    plsc.subcore_barrier()

    nout = ROWS_PER_TILE // K

    def osl(i):
        return pl.ds(r0 + i * K, K)

    for i in range(nout):
        b = i % 2
        if i >= 2:
            pltpu.make_async_copy(rows[b], out_hbm.at[c, osl(i - 2)],
                                  sems[b]).wait()
        pltpu.sync_copy(acc_sp.at[osl(i)], rows[b])
        pltpu.async_copy(rows[b], out_hbm.at[c, osl(i)], sems[b])
    for i in range(max(nout - 2, 0), nout):
        b = i % 2
        pltpu.make_async_copy(rows[b], out_hbm.at[c, osl(i)], sems[b]).wait()


@functools.cache
def _build_scatter(dd):
    return functools.partial(
        pl.kernel,
        out_type=jax.ShapeDtypeStruct((NC, NPAD, dd), jnp.float32),
        mesh=_sc_mesh(),
        compiler_params=pltpu.CompilerParams(needs_layout_passes=False),
        scratch_types=[
            pltpu.VMEM_SHARED((NPAD, dd), jnp.float32),
            pltpu.VMEM((SUP, K), jnp.int32),
            pltpu.VMEM((SUP, K), jnp.int32),
            pltpu.VMEM((K, dd), jnp.float32),
            pltpu.VMEM((K, dd), jnp.float32),
            pltpu.SemaphoreType.DMA,
            pltpu.SemaphoreType.DMA,
            pltpu.SemaphoreType.DMA,
            pltpu.SemaphoreType.DMA,
        ],
    )(_scatter_body)


def _dis(degt_ref):
    deg = degt_ref[...]
    degs = deg[:, 0:1] + deg[:, 1:2]
    return jnp.where(degs > 0, lax.rsqrt(jnp.maximum(degs, 1e-12)), 0.0)


def _mm1_body(x_ref, w1_ref, degt_ref, xw_ref, y_ref):
    xw = jnp.dot(x_ref[...], w1_ref[...], preferred_element_type=jnp.float32)
    xw_ref[...] = xw
    y_ref[...] = xw * _dis(degt_ref)


def _mid_body(a0_ref, a1_ref, degt_ref, xw1_ref, b1_ref, w2_ref,
              xw2_ref, y2_ref):
    dis = _dis(degt_ref)
    h1 = (a0_ref[...] + a1_ref[...]) * dis + b1_ref[...]
    h12 = xw1_ref[...] + b1_ref[...]
    r1a = jnp.maximum(h1, 0.0)
    r1b = jnp.maximum(h12, 0.0)
    w2 = w2_ref[...]
    xw2 = (jnp.dot(r1a, w2[:H], preferred_element_type=jnp.float32)
           + jnp.dot(r1b, w2[H:], preferred_element_type=jnp.float32))
    xw2_ref[...] = xw2
    y2_ref[...] = jnp.concatenate([xw2 * dis, jnp.zeros_like(xw2)], axis=1)


def _fin_body(c0_ref, c1_ref, degt_ref, xw2_ref, b2_ref, wl_ref, bl_ref,
              out_ref):
    dis = _dis(degt_ref)
    h2 = (c0_ref[:, :C] + c1_ref[:, :C]) * dis + b2_ref[...]
    h22 = xw2_ref[...] + b2_ref[...]
    wl = wl_ref[...]
    f = (jnp.dot(h2, wl[:C], preferred_element_type=jnp.float32)
         + jnp.dot(h22, wl[C:], preferred_element_type=jnp.float32)
         + bl_ref[...])
    m = jnp.max(f, axis=1, keepdims=True)
    e = jnp.exp(f - m)
    out_ref[...] = f - m - jnp.log(jnp.sum(e, axis=1, keepdims=True))


def _row_spec(cols):
    return pl.BlockSpec((BR, cols), lambda i: (i, 0))


def _full_spec(r, cols):
    return pl.BlockSpec((r, cols), lambda i: (0, 0))


_GRID = (NPAD // BR,)

_mm1 = pl.pallas_call(
    _mm1_body,
    grid=_GRID,
    in_specs=[_row_spec(D), _full_spec(D, H), _row_spec(2)],
    out_specs=[_row_spec(H), _row_spec(H)],
    out_shape=[jax.ShapeDtypeStruct((NPAD, H), jnp.float32)] * 2,
)

_mid = pl.pallas_call(
    _mid_body,
    grid=_GRID,
    in_specs=[_row_spec(H), _row_spec(H), _row_spec(2), _row_spec(H),
              _full_spec(1, H), _full_spec(2 * H, C)],
    out_specs=[_row_spec(C), _row_spec(H)],
    out_shape=[jax.ShapeDtypeStruct((NPAD, C), jnp.float32),
               jax.ShapeDtypeStruct((NPAD, H), jnp.float32)],
)

_fin = pl.pallas_call(
    _fin_body,
    grid=_GRID,
    in_specs=[_row_spec(H), _row_spec(H), _row_spec(2), _row_spec(C),
              _full_spec(1, C), _full_spec(2 * C, C), _full_spec(1, C)],
    out_specs=_row_spec(C),
    out_shape=jax.ShapeDtypeStruct((NPAD, C), jnp.float32),
)


def kernel(x, edge_index, W1, b1, W2, b2, Wlin, blin):
    row = edge_index[0]
    col = edge_index[1]

    degp = _build_prep()(row, col)
    degt = degp.T

    pad2 = ((0, NCHUNKS_PAD - NCHUNKS), (0, 0))
    row2 = jnp.pad(row.reshape(NCHUNKS, K), pad2)
    col2 = jnp.pad(col.reshape(NCHUNKS, K), pad2)
    zr = jnp.zeros((K, H), jnp.float32)
    xw1, y1 = _mm1(x, W1, degt)
    acc1 = _build_scatter(H)(y1, row2, col2, zr)
    xw2, y2 = _mid(acc1[0], acc1[1], degt, xw1, b1.reshape(1, H), W2)
    acc2 = _build_scatter(H)(y2, row2, col2, zr)
    outp = _fin(acc2[0], acc2[1], degt, xw2, b2.reshape(1, C),
                Wlin, blin.reshape(1, C))
    return outp[:N]


# speedup vs baseline: 1.2191x; 1.2191x over previous
import functools

import jax
import jax.numpy as jnp
from jax import lax
from jax.experimental import pallas as pl
from jax.experimental.pallas import tpu as pltpu
from jax.experimental.pallas import tpu_sc as plsc

N = 10000
E = 320000
D = 128
H = 128
C = 64

NC = 2
NS = 16
NW = NC * NS
LANES = 16

NPAD = 10240
K = 128
NCHUNKS = E // K
CHUNKS_PER_W = -(-NCHUNKS // NW)
CMAX = 80
NCHUNKS_PAD = NW * CMAX
SUP = 40
KP = 400
EPW = E // NW
PBLOCKS = EPW // KP
ROWS_PER_TILE = NPAD // NS
BR = 1024


def _wid():
    c = lax.axis_index("c")
    s = lax.axis_index("s")
    return s * NC + c, c, s


def _prep_body(row_hbm, col_hbm, degp_hbm,
               degall_sp, row_a, col_a, row_b, col_b, sema, semb,
               deg_v, tmp_v, acc_v):
    wid, c, s = _wid()
    zeros16 = jnp.zeros((LANES,), jnp.float32)
    ones16 = jnp.ones((LANES,), jnp.float32)
    e0 = wid * EPW
    bufs = [(row_a, col_a, sema), (row_b, col_b, semb)]

    def fire(b, rv, cv, sem):
        @pl.when(b < PBLOCKS)
        def _():
            base = e0 + b * KP
            pltpu.async_copy(row_hbm.at[pl.ds(base, KP)], rv, sem)
            pltpu.async_copy(col_hbm.at[pl.ds(base, KP)], cv, sem)

    def consume(b, rv, cv, sem):
        @pl.when(b < PBLOCKS)
        def _():
            base = e0 + b * KP
            pltpu.make_async_copy(row_hbm.at[pl.ds(base, KP)], rv, sem).wait()
            pltpu.make_async_copy(col_hbm.at[pl.ds(base, KP)], cv, sem).wait()
            for j in range(KP // LANES):
                sl = pl.ds(j * LANES, LANES)
                r = rv[sl]
                co = cv[sl]
                cp = jnp.where(r == co, N, co)
                plsc.addupdate_scatter(deg_v, [cp], ones16)

    @pl.loop(0, NPAD // LANES)
    def _(j):
        deg_v[pl.ds(j * LANES, LANES)] = zeros16

    fire(0, *bufs[0])

    @pl.loop(0, PBLOCKS + 1, step=2)
    def _(t):
        fire(t + 1, *bufs[1])
        consume(t, *bufs[0])
        fire(t + 2, *bufs[0])
        consume(t + 1, *bufs[1])

    pltpu.sync_copy(deg_v, degall_sp.at[s])
    plsc.subcore_barrier()

    @pl.loop(0, ROWS_PER_TILE // LANES)
    def _(j):
        acc_v[pl.ds(j * LANES, LANES)] = zeros16

    @pl.loop(0, NS)
    def _(t):
        pltpu.sync_copy(degall_sp.at[t, pl.ds(s * ROWS_PER_TILE, ROWS_PER_TILE)],
                        tmp_v)

        @pl.loop(0, ROWS_PER_TILE // LANES)
        def _(j):
            sl = pl.ds(j * LANES, LANES)
            acc_v[sl] = acc_v[sl] + tmp_v[sl]

    pltpu.sync_copy(acc_v, degp_hbm.at[c, pl.ds(s * ROWS_PER_TILE, ROWS_PER_TILE)])


def _sc_mesh():
    return plsc.VectorSubcoreMesh(core_axis_name="c", subcore_axis_name="s",
                                  num_cores=NC, num_subcores=NS)


@functools.cache
def _build_prep():
    return functools.partial(
        pl.kernel,
        out_type=jax.ShapeDtypeStruct((NC, NPAD), jnp.float32),
        mesh=_sc_mesh(),
        compiler_params=pltpu.CompilerParams(needs_layout_passes=False),
        scratch_types=[
            pltpu.VMEM_SHARED((NS, NPAD), jnp.float32),
            pltpu.VMEM((KP,), jnp.int32),
            pltpu.VMEM((KP,), jnp.int32),
            pltpu.VMEM((KP,), jnp.int32),
            pltpu.VMEM((KP,), jnp.int32),
            pltpu.SemaphoreType.DMA,
            pltpu.SemaphoreType.DMA,
            pltpu.VMEM((NPAD,), jnp.float32),
            pltpu.VMEM((ROWS_PER_TILE,), jnp.float32),
            pltpu.VMEM((ROWS_PER_TILE,), jnp.float32),
        ],
    )(_prep_body)


def _scatter_body(y_hbm, row2_hbm, col2_hbm, zero_hbm, out_hbm,
                  acc_sp, ridx2, cidx2, rows0, rows1, sem0, sem1):
    wid, c, s = _wid()
    r0 = s * ROWS_PER_TILE
    rows = [rows0, rows1]
    sems = [sem0, sem1]
    start = CMAX * wid
    nch = jnp.minimum(CMAX, NCHUNKS - start)

    pltpu.sync_copy(zero_hbm, rows0)
    for i in range(ROWS_PER_TILE // K):
        pltpu.async_copy(rows0, acc_sp.at[pl.ds(r0 + i * K, K)], sem0)
    for i in range(ROWS_PER_TILE // K):
        pltpu.make_async_copy(rows0, acc_sp.at[pl.ds(r0 + i * K, K)],
                              sem0).wait()

    plsc.subcore_barrier()

    KH = K // 2

    def fire(u, j, b):
        q = u * SUP + j

        @pl.when(q < nch)
        def _():
            pltpu.async_copy(y_hbm.at[ridx2.at[j, pl.ds(0, KH)]],
                             rows[b].at[pl.ds(0, KH)], sems[b])
            pltpu.async_copy(y_hbm.at[ridx2.at[j, pl.ds(KH, KH)]],
                             rows[b].at[pl.ds(KH, KH)], sems[b])

    def consume(u, j, b):
        q = u * SUP + j

        @pl.when(q < nch)
        def _():
            pltpu.make_async_copy(y_hbm.at[ridx2.at[j, pl.ds(0, KH)]],
                                  rows[b].at[pl.ds(0, KH)], sems[b]).wait()
            pltpu.make_async_copy(y_hbm.at[ridx2.at[j, pl.ds(KH, KH)]],
                                  rows[b].at[pl.ds(KH, KH)], sems[b]).wait()
            pltpu.sync_copy(rows[b], acc_sp.at[cidx2.at[j]], add=True)

    @pl.loop(0, CMAX // SUP)
    def _(u):
        @pl.when(u * SUP < nch)
        def _():
            sl_u = pl.ds(start + u * SUP, SUP)
            pltpu.sync_copy(row2_hbm.at[sl_u], ridx2)
            pltpu.sync_copy(col2_hbm.at[sl_u], cidx2)

            @pl.loop(0, SUP)
            def _(q):
                for j in range(K // LANES):
                    sl = pl.ds(j * LANES, LANES)
                    r = ridx2[q, sl]
                    cv = cidx2[q, sl]
                    cidx2[q, sl] = jnp.where(r == cv, N, cv)

        fire(u, 0, 0)
        fire(u, 1, 1)
        for j in range(0, SUP - 2, 2):
            consume(u, j, 0)
            fire(u, j + 2, 0)
            consume(u, j + 1, 1)
            fire(u, j + 3, 1)
        consume(u, SUP - 2, 0)
        consume(u, SUP - 1, 1)

    plsc.subcore_barrier()

    nout = ROWS_PER_TILE // K

    def osl(i):
        return pl.ds(r0 + i * K, K)

    for i in range(nout):
        b = i % 2
        if i >= 2:
            pltpu.make_async_copy(rows[b], out_hbm.at[c, osl(i - 2)],
                                  sems[b]).wait()
        pltpu.sync_copy(acc_sp.at[osl(i)], rows[b])
        pltpu.async_copy(rows[b], out_hbm.at[c, osl(i)], sems[b])
    for i in range(max(nout - 2, 0), nout):
        b = i % 2
        pltpu.make_async_copy(rows[b], out_hbm.at[c, osl(i)], sems[b]).wait()


@functools.cache
def _build_scatter(dd):
    return functools.partial(
        pl.kernel,
        out_type=jax.ShapeDtypeStruct((NC, NPAD, dd), jnp.float32),
        mesh=_sc_mesh(),
        compiler_params=pltpu.CompilerParams(needs_layout_passes=False),
        scratch_types=[
            pltpu.VMEM_SHARED((NPAD, dd), jnp.float32),
            pltpu.VMEM((SUP, K), jnp.int32),
            pltpu.VMEM((SUP, K), jnp.int32),
            pltpu.VMEM((K, dd), jnp.float32),
            pltpu.VMEM((K, dd), jnp.float32),
            pltpu.SemaphoreType.DMA,
            pltpu.SemaphoreType.DMA,
        ],
    )(_scatter_body)


def _dis(degt_ref):
    deg = degt_ref[...]
    degs = deg[:, 0:1] + deg[:, 1:2]
    return jnp.where(degs > 0, lax.rsqrt(jnp.maximum(degs, 1e-12)), 0.0)


def _mm1_body(x_ref, w1_ref, degt_ref, xw_ref, y_ref):
    xw = jnp.dot(x_ref[...], w1_ref[...], preferred_element_type=jnp.float32)
    xw_ref[...] = xw
    y_ref[...] = xw * _dis(degt_ref)


def _mid_body(a0_ref, a1_ref, degt_ref, xw1_ref, b1_ref, w2_ref,
              xw2_ref, y2_ref):
    dis = _dis(degt_ref)
    h1 = (a0_ref[...] + a1_ref[...]) * dis + b1_ref[...]
    h12 = xw1_ref[...] + b1_ref[...]
    r1a = jnp.maximum(h1, 0.0)
    r1b = jnp.maximum(h12, 0.0)
    w2 = w2_ref[...]
    xw2 = (jnp.dot(r1a, w2[:H], preferred_element_type=jnp.float32)
           + jnp.dot(r1b, w2[H:], preferred_element_type=jnp.float32))
    xw2_ref[...] = xw2
    y2_ref[...] = jnp.concatenate([xw2 * dis, jnp.zeros_like(xw2)], axis=1)


def _fin_body(c0_ref, c1_ref, degt_ref, xw2_ref, b2_ref, wl_ref, bl_ref,
              out_ref):
    dis = _dis(degt_ref)
    h2 = (c0_ref[:, :C] + c1_ref[:, :C]) * dis + b2_ref[...]
    h22 = xw2_ref[...] + b2_ref[...]
    wl = wl_ref[...]
    f = (jnp.dot(h2, wl[:C], preferred_element_type=jnp.float32)
         + jnp.dot(h22, wl[C:], preferred_element_type=jnp.float32)
         + bl_ref[...])
    m = jnp.max(f, axis=1, keepdims=True)
    e = jnp.exp(f - m)
    out_ref[...] = f - m - jnp.log(jnp.sum(e, axis=1, keepdims=True))


def _row_spec(cols):
    return pl.BlockSpec((BR, cols), lambda i: (i, 0))


def _full_spec(r, cols):
    return pl.BlockSpec((r, cols), lambda i: (0, 0))


_GRID = (NPAD // BR,)

_mm1 = pl.pallas_call(
    _mm1_body,
    grid=_GRID,
    in_specs=[_row_spec(D), _full_spec(D, H), _row_spec(2)],
    out_specs=[_row_spec(H), _row_spec(H)],
    out_shape=[jax.ShapeDtypeStruct((NPAD, H), jnp.float32)] * 2,
)

_mid = pl.pallas_call(
    _mid_body,
    grid=_GRID,
    in_specs=[_row_spec(H), _row_spec(H), _row_spec(2), _row_spec(H),
              _full_spec(1, H), _full_spec(2 * H, C)],
    out_specs=[_row_spec(C), _row_spec(H)],
    out_shape=[jax.ShapeDtypeStruct((NPAD, C), jnp.float32),
               jax.ShapeDtypeStruct((NPAD, H), jnp.float32)],
)

_fin = pl.pallas_call(
    _fin_body,
    grid=_GRID,
    in_specs=[_row_spec(H), _row_spec(H), _row_spec(2), _row_spec(C),
              _full_spec(1, C), _full_spec(2 * C, C), _full_spec(1, C)],
    out_specs=_row_spec(C),
    out_shape=jax.ShapeDtypeStruct((NPAD, C), jnp.float32),
)


def kernel(x, edge_index, W1, b1, W2, b2, Wlin, blin):
    row = edge_index[0]
    col = edge_index[1]

    degp = _build_prep()(row, col)
    degt = degp.T

    pad2 = ((0, NCHUNKS_PAD - NCHUNKS), (0, 0))
    row2 = jnp.pad(row.reshape(NCHUNKS, K), pad2)
    col2 = jnp.pad(col.reshape(NCHUNKS, K), pad2)
    zr = jnp.zeros((K, H), jnp.float32)
    xw1, y1 = _mm1(x, W1, degt)
    acc1 = _build_scatter(H)(y1, row2, col2, zr)
    xw2, y2 = _mid(acc1[0], acc1[1], degt, xw1, b1.reshape(1, H), W2)
    acc2 = _build_scatter(H)(y2, row2, col2, zr)
    outp = _fin(acc2[0], acc2[1], degt, xw2, b2.reshape(1, C),
                Wlin, blin.reshape(1, C))
    return outp[:N]


# speedup vs baseline: 1.2311x; 1.0098x over previous
import functools

import jax
import jax.numpy as jnp
from jax import lax
from jax.experimental import pallas as pl
from jax.experimental.pallas import tpu as pltpu
from jax.experimental.pallas import tpu_sc as plsc

N = 10000
E = 320000
D = 128
H = 128
C = 64

NC = 2
NS = 16
NW = NC * NS
LANES = 16

NPAD = 10240
K = 128
NCHUNKS = E // K
CHUNKS_PER_W = -(-NCHUNKS // NW)
CMAX = 80
NCHUNKS_PAD = NW * CMAX
SUP = 40
KP = 400
EPW = E // NW
PBLOCKS = EPW // KP
ROWS_PER_TILE = NPAD // NS
BR = 1024


def _wid():
    c = lax.axis_index("c")
    s = lax.axis_index("s")
    return s * NC + c, c, s


def _prep_body(row_hbm, col_hbm, degp_hbm,
               degall_sp, row_a, col_a, row_b, col_b, sema, semb,
               deg_v, tmp_v, tmp2_v, acc_v):
    wid, c, s = _wid()
    zeros16 = jnp.zeros((LANES,), jnp.float32)
    ones16 = jnp.ones((LANES,), jnp.float32)
    e0 = wid * EPW
    bufs = [(row_a, col_a, sema), (row_b, col_b, semb)]

    def fire(b, rv, cv, sem):
        @pl.when(b < PBLOCKS)
        def _():
            base = e0 + b * KP
            pltpu.async_copy(row_hbm.at[pl.ds(base, KP)], rv, sem)
            pltpu.async_copy(col_hbm.at[pl.ds(base, KP)], cv, sem)

    def consume(b, rv, cv, sem):
        @pl.when(b < PBLOCKS)
        def _():
            base = e0 + b * KP
            pltpu.make_async_copy(row_hbm.at[pl.ds(base, KP)], rv, sem).wait()
            pltpu.make_async_copy(col_hbm.at[pl.ds(base, KP)], cv, sem).wait()
            for j in range(KP // LANES):
                sl = pl.ds(j * LANES, LANES)
                r = rv[sl]
                co = cv[sl]
                cp = jnp.where(r == co, N, co)
                plsc.addupdate_scatter(deg_v, [cp], ones16)

    @pl.loop(0, NPAD // (8 * LANES))
    def _(j):
        for i in range(8):
            deg_v[pl.ds((j * 8 + i) * LANES, LANES)] = zeros16

    fire(0, *bufs[0])

    @pl.loop(0, PBLOCKS + 1, step=2)
    def _(t):
        fire(t + 1, *bufs[1])
        consume(t, *bufs[0])
        fire(t + 2, *bufs[0])
        consume(t + 1, *bufs[1])

    pltpu.sync_copy(deg_v, degall_sp.at[s])
    plsc.subcore_barrier()
    rsl = pl.ds(s * ROWS_PER_TILE, ROWS_PER_TILE)

    @pl.loop(0, ROWS_PER_TILE // LANES)
    def _(j):
        acc_v[pl.ds(j * LANES, LANES)] = zeros16

    def tfire(t, tv, tsem):
        if t < NS:
            pltpu.async_copy(degall_sp.at[t, rsl], tv, tsem)

    def tconsume(t, tv, tsem):
        if t < NS:
            pltpu.make_async_copy(degall_sp.at[t, rsl], tv, tsem).wait()

            @pl.loop(0, ROWS_PER_TILE // (8 * LANES))
            def _(j):
                for i in range(8):
                    sl = pl.ds((j * 8 + i) * LANES, LANES)
                    acc_v[sl] = acc_v[sl] + tv[sl]

    tfire(0, tmp_v, sema)
    for t in range(0, NS, 2):
        tfire(t + 1, tmp2_v, semb)
        tconsume(t, tmp_v, sema)
        tfire(t + 2, tmp_v, sema)
        tconsume(t + 1, tmp2_v, semb)

    pltpu.sync_copy(acc_v, degp_hbm.at[c, rsl])


def _sc_mesh():
    return plsc.VectorSubcoreMesh(core_axis_name="c", subcore_axis_name="s",
                                  num_cores=NC, num_subcores=NS)


@functools.cache
def _build_prep():
    return functools.partial(
        pl.kernel,
        out_type=jax.ShapeDtypeStruct((NC, NPAD), jnp.float32),
        mesh=_sc_mesh(),
        compiler_params=pltpu.CompilerParams(needs_layout_passes=False),
        scratch_types=[
            pltpu.VMEM_SHARED((NS, NPAD), jnp.float32),
            pltpu.VMEM((KP,), jnp.int32),
            pltpu.VMEM((KP,), jnp.int32),
            pltpu.VMEM((KP,), jnp.int32),
            pltpu.VMEM((KP,), jnp.int32),
            pltpu.SemaphoreType.DMA,
            pltpu.SemaphoreType.DMA,
            pltpu.VMEM((NPAD,), jnp.float32),
            pltpu.VMEM((ROWS_PER_TILE,), jnp.float32),
            pltpu.VMEM((ROWS_PER_TILE,), jnp.float32),
            pltpu.VMEM((ROWS_PER_TILE,), jnp.float32),
        ],
    )(_prep_body)


def _scatter_body(y_hbm, row2_hbm, col2_hbm, zero_hbm, out_hbm,
                  acc_sp, ridx2, cidx2, rows0, rows1, sem0, sem1):
    wid, c, s = _wid()
    r0 = s * ROWS_PER_TILE
    rows = [rows0, rows1]
    sems = [sem0, sem1]
    start = CMAX * wid
    nch = jnp.minimum(CMAX, NCHUNKS - start)

    pltpu.sync_copy(zero_hbm, rows0)
    for i in range(ROWS_PER_TILE // K):
        pltpu.async_copy(rows0, acc_sp.at[pl.ds(r0 + i * K, K)], sem0)
    for i in range(ROWS_PER_TILE // K):
        pltpu.make_async_copy(rows0, acc_sp.at[pl.ds(r0 + i * K, K)],
                              sem0).wait()

    plsc.subcore_barrier()

    KH = K // 2

    def fire(u, j, b):
        q = u * SUP + j

        @pl.when(q < nch)
        def _():
            pltpu.async_copy(y_hbm.at[ridx2.at[j, pl.ds(0, KH)]],
                             rows[b].at[pl.ds(0, KH)], sems[b])
            pltpu.async_copy(y_hbm.at[ridx2.at[j, pl.ds(KH, KH)]],
                             rows[b].at[pl.ds(KH, KH)], sems[b])

    def consume(u, j, b):
        q = u * SUP + j

        @pl.when(q < nch)
        def _():
            pltpu.make_async_copy(y_hbm.at[ridx2.at[j, pl.ds(0, KH)]],
                                  rows[b].at[pl.ds(0, KH)], sems[b]).wait()
            pltpu.make_async_copy(y_hbm.at[ridx2.at[j, pl.ds(KH, KH)]],
                                  rows[b].at[pl.ds(KH, KH)], sems[b]).wait()
            pltpu.sync_copy(rows[b], acc_sp.at[cidx2.at[j]], add=True)

    @pl.loop(0, CMAX // SUP)
    def _(u):
        @pl.when(u * SUP < nch)
        def _():
            sl_u = pl.ds(start + u * SUP, SUP)
            pltpu.sync_copy(row2_hbm.at[sl_u], ridx2)
            pltpu.sync_copy(col2_hbm.at[sl_u], cidx2)

            @pl.loop(0, SUP)
            def _(q):
                for j in range(K // LANES):
                    sl = pl.ds(j * LANES, LANES)
                    r = ridx2[q, sl]
                    cv = cidx2[q, sl]
                    cidx2[q, sl] = jnp.where(r == cv, N, cv)

        fire(u, 0, 0)
        fire(u, 1, 1)
        for j in range(0, SUP - 2, 2):
            consume(u, j, 0)
            fire(u, j + 2, 0)
            consume(u, j + 1, 1)
            fire(u, j + 3, 1)
        consume(u, SUP - 2, 0)
        consume(u, SUP - 1, 1)

    plsc.subcore_barrier()

    nout = ROWS_PER_TILE // K

    def osl(i):
        return pl.ds(r0 + i * K, K)

    for i in range(nout):
        b = i % 2
        if i >= 2:
            pltpu.make_async_copy(rows[b], out_hbm.at[c, osl(i - 2)],
                                  sems[b]).wait()
        pltpu.sync_copy(acc_sp.at[osl(i)], rows[b])
        pltpu.async_copy(rows[b], out_hbm.at[c, osl(i)], sems[b])
    for i in range(max(nout - 2, 0), nout):
        b = i % 2
        pltpu.make_async_copy(rows[b], out_hbm.at[c, osl(i)], sems[b]).wait()


@functools.cache
def _build_scatter(dd):
    return functools.partial(
        pl.kernel,
        out_type=jax.ShapeDtypeStruct((NC, NPAD, dd), jnp.float32),
        mesh=_sc_mesh(),
        compiler_params=pltpu.CompilerParams(needs_layout_passes=False),
        scratch_types=[
            pltpu.VMEM_SHARED((NPAD, dd), jnp.float32),
            pltpu.VMEM((SUP, K), jnp.int32),
            pltpu.VMEM((SUP, K), jnp.int32),
            pltpu.VMEM((K, dd), jnp.float32),
            pltpu.VMEM((K, dd), jnp.float32),
            pltpu.SemaphoreType.DMA,
            pltpu.SemaphoreType.DMA,
        ],
    )(_scatter_body)


def _dis(degt_ref):
    deg = degt_ref[...]
    degs = deg[:, 0:1] + deg[:, 1:2]
    return jnp.where(degs > 0, lax.rsqrt(jnp.maximum(degs, 1e-12)), 0.0)


def _mm1_body(x_ref, w1_ref, degt_ref, xw_ref, y_ref):
    xw = jnp.dot(x_ref[...], w1_ref[...], preferred_element_type=jnp.float32)
    xw_ref[...] = xw
    y_ref[...] = xw * _dis(degt_ref)


def _mid_body(a0_ref, a1_ref, degt_ref, xw1_ref, b1_ref, w2_ref,
              xw2_ref, y2_ref):
    dis = _dis(degt_ref)
    h1 = (a0_ref[...] + a1_ref[...]) * dis + b1_ref[...]
    h12 = xw1_ref[...] + b1_ref[...]
    r1a = jnp.maximum(h1, 0.0)
    r1b = jnp.maximum(h12, 0.0)
    w2 = w2_ref[...]
    xw2 = (jnp.dot(r1a, w2[:H], preferred_element_type=jnp.float32)
           + jnp.dot(r1b, w2[H:], preferred_element_type=jnp.float32))
    xw2_ref[...] = xw2
    y2_ref[...] = jnp.concatenate([xw2 * dis, jnp.zeros_like(xw2)], axis=1)


def _fin_body(c0_ref, c1_ref, degt_ref, xw2_ref, b2_ref, wl_ref, bl_ref,
              out_ref):
    dis = _dis(degt_ref)
    h2 = (c0_ref[:, :C] + c1_ref[:, :C]) * dis + b2_ref[...]
    h22 = xw2_ref[...] + b2_ref[...]
    wl = wl_ref[...]
    f = (jnp.dot(h2, wl[:C], preferred_element_type=jnp.float32)
         + jnp.dot(h22, wl[C:], preferred_element_type=jnp.float32)
         + bl_ref[...])
    m = jnp.max(f, axis=1, keepdims=True)
    e = jnp.exp(f - m)
    out_ref[...] = f - m - jnp.log(jnp.sum(e, axis=1, keepdims=True))


def _row_spec(cols):
    return pl.BlockSpec((BR, cols), lambda i: (i, 0))


def _full_spec(r, cols):
    return pl.BlockSpec((r, cols), lambda i: (0, 0))


_GRID = (NPAD // BR,)

_mm1 = pl.pallas_call(
    _mm1_body,
    grid=_GRID,
    in_specs=[_row_spec(D), _full_spec(D, H), _row_spec(2)],
    out_specs=[_row_spec(H), _row_spec(H)],
    out_shape=[jax.ShapeDtypeStruct((NPAD, H), jnp.float32)] * 2,
)

_mid = pl.pallas_call(
    _mid_body,
    grid=_GRID,
    in_specs=[_row_spec(H), _row_spec(H), _row_spec(2), _row_spec(H),
              _full_spec(1, H), _full_spec(2 * H, C)],
    out_specs=[_row_spec(C), _row_spec(H)],
    out_shape=[jax.ShapeDtypeStruct((NPAD, C), jnp.float32),
               jax.ShapeDtypeStruct((NPAD, H), jnp.float32)],
)

_fin = pl.pallas_call(
    _fin_body,
    grid=_GRID,
    in_specs=[_row_spec(H), _row_spec(H), _row_spec(2), _row_spec(C),
              _full_spec(1, C), _full_spec(2 * C, C), _full_spec(1, C)],
    out_specs=_row_spec(C),
    out_shape=jax.ShapeDtypeStruct((NPAD, C), jnp.float32),
)


def kernel(x, edge_index, W1, b1, W2, b2, Wlin, blin):
    row = edge_index[0]
    col = edge_index[1]

    degp = _build_prep()(row, col)
    degt = degp.T

    pad2 = ((0, NCHUNKS_PAD - NCHUNKS), (0, 0))
    row2 = jnp.pad(row.reshape(NCHUNKS, K), pad2)
    col2 = jnp.pad(col.reshape(NCHUNKS, K), pad2)
    zr = jnp.zeros((K, H), jnp.float32)
    xw1, y1 = _mm1(x, W1, degt)
    acc1 = _build_scatter(H)(y1, row2, col2, zr)
    xw2, y2 = _mid(acc1[0], acc1[1], degt, xw1, b1.reshape(1, H), W2)
    acc2 = _build_scatter(H)(y2, row2, col2, zr)
    outp = _fin(acc2[0], acc2[1], degt, xw2, b2.reshape(1, C),
                Wlin, blin.reshape(1, C))
    return outp[:N]
